# Initial kernel scaffold; baseline (speedup 1.0000x reference)
import jax
import jax.numpy as jnp
from jax.experimental import pallas as pl


def kernel(x, timestamps, ht_positions, token):
    raise NotImplementedError("write your pallas kernel here")


# speedup vs baseline: 6.3362x; 6.3362x over previous
import functools

import jax
import jax.numpy as jnp
from jax import lax
from jax.experimental import pallas as pl
from jax.experimental.pallas import tpu as pltpu
from jax.experimental.pallas import tpu_sc as plsc


def _meta_body(htp_smem, htp_ref, oidx_ref, htg_ref, cq_ref, isht_ref,
               *, L, R, n):
    b = pl.program_id(0)
    j_row = lax.broadcasted_iota(jnp.int32, (1, L), 1)
    q_row = lax.broadcasted_iota(jnp.int32, (1, n), 1)

    def body(i, carry):
        cnt_lt, c, isht = carry
        p = htp_smem[b, i]
        slot = p + i + 1
        cnt_lt = cnt_lt + (j_row > p).astype(jnp.int32)
        c = c + (q_row >= slot).astype(jnp.int32)
        isht = isht + (q_row == slot).astype(jnp.int32)
        return cnt_lt, c, isht

    zL = jnp.zeros((1, L), jnp.int32)
    zn = jnp.zeros((1, n), jnp.int32)
    cnt_lt, c, isht = lax.fori_loop(0, R, body, (zL, zn, zn))

    oidx_ref[0] = j_row + cnt_lt + b * n
    r_iota = lax.broadcasted_iota(jnp.int32, (1, R), 1)
    htg_ref[0] = htp_ref[0] + r_iota + 1 + b * n
    cq_ref[0] = c
    isht_ref[0] = isht


def _run_meta(htp, B, L, R, n):
    grid = (B,)
    out_shapes = (
        jax.ShapeDtypeStruct((B, 1, L), jnp.int32),
        jax.ShapeDtypeStruct((B, 1, R), jnp.int32),
        jax.ShapeDtypeStruct((B, 1, n), jnp.int32),
        jax.ShapeDtypeStruct((B, 1, n), jnp.int32),
    )
    return pl.pallas_call(
        functools.partial(_meta_body, L=L, R=R, n=n),
        grid=grid,
        in_specs=[
            pl.BlockSpec(memory_space=pltpu.SMEM),
            pl.BlockSpec((1, 1, R), lambda b: (b, 0, 0)),
        ],
        out_specs=(
            pl.BlockSpec((1, 1, L), lambda b: (b, 0, 0)),
            pl.BlockSpec((1, 1, R), lambda b: (b, 0, 0)),
            pl.BlockSpec((1, 1, n), lambda b: (b, 0, 0)),
            pl.BlockSpec((1, 1, n), lambda b: (b, 0, 0)),
        ),
        out_shape=out_shapes,
    )(htp, htp.reshape(B, 1, R))


def _mask_body(htp_ref, cq_ref, isht_ref, out_ref, *, R, n, PR):
    j = pl.program_id(1)
    r_iota = lax.broadcasted_iota(jnp.int32, (1, R), 1)
    htslot = htp_ref[0] + r_iota + 1
    p_col = lax.broadcasted_iota(jnp.int32, (PR, 1), 0) + j * PR
    le = htslot <= p_col
    c_p = jnp.sum(le.astype(jnp.int32), axis=1, keepdims=True)
    isht_p = jnp.sum((htslot == p_col).astype(jnp.int32), axis=1,
                     keepdims=True)
    H_p = jnp.max(jnp.where(le, jnp.broadcast_to(htslot, (PR, R)), 0),
                  axis=1, keepdims=True)
    H_p = jnp.where(isht_p > 0, 0, H_p)

    q_row = lax.broadcasted_iota(jnp.int32, (1, n), 1)
    c_q = cq_ref[0]
    isht_q = isht_ref[0]
    ne = jnp.minimum(jnp.abs(c_p - c_q), 1)
    lt = jnp.clip(H_p - q_row, 0, 1)
    out32 = isht_q * ne + (1 - isht_q) * lt
    out_ref[0] = out32.astype(jnp.int8)


def _run_mask(htp3, cq, isht, B, R, n, PR):
    grid = (B, n // PR)
    return pl.pallas_call(
        functools.partial(_mask_body, R=R, n=n, PR=PR),
        grid=grid,
        in_specs=[
            pl.BlockSpec((1, 1, R), lambda b, j: (b, 0, 0)),
            pl.BlockSpec((1, 1, n), lambda b, j: (b, 0, 0)),
            pl.BlockSpec((1, 1, n), lambda b, j: (b, 0, 0)),
        ],
        out_specs=pl.BlockSpec((1, PR, n), lambda b, j: (b, j, 0)),
        out_shape=jax.ShapeDtypeStruct((B, n, n), jnp.int8),
    )(htp3, cq, isht)


_ROWS_CHUNK = 32
_TS_CHUNK = 128


def _sc_body(x2, tsf, ts128, oidx32, oidx128, htg, htpf, tok_rows,
             outx2, outts, idxbuf, rowbuf, tsidx, tsbuf, tokbuf,
             htgv, htpv, httsv, sem, *, L, D, n, NW, HT_PER_W):
    c_i = lax.axis_index("c")
    s_i = lax.axis_index("s")
    w = s_i * 2 + c_i
    rows_per_w = x2.shape[0] // NW
    nch = rows_per_w // _ROWS_CHUNK
    wbase = w * rows_per_w

    pltpu.sync_copy(oidx32.at[pl.ds(w * nch, nch)], idxbuf)
    for j in range(nch):
        pltpu.sync_copy(x2.at[pl.ds(wbase + j * _ROWS_CHUNK, _ROWS_CHUNK)],
                        rowbuf)
        pltpu.async_copy(rowbuf, outx2.at[idxbuf.at[j]], sem).wait()

    tch = rows_per_w // _TS_CHUNK
    pltpu.sync_copy(oidx128.at[pl.ds(w * tch, tch)], tsidx)
    pltpu.sync_copy(ts128.at[pl.ds(w * tch, tch)], tsbuf)
    for j in range(tch):
        pltpu.async_copy(tsbuf.at[j], outts.at[tsidx.at[j]], sem).wait()

    pltpu.sync_copy(htg.at[pl.ds(w * HT_PER_W, HT_PER_W)], htgv)
    pltpu.sync_copy(tok_rows, tokbuf)
    pltpu.async_copy(tokbuf, outx2.at[htgv], sem).wait()

    pltpu.sync_copy(htpf.at[pl.ds(w * HT_PER_W, HT_PER_W)], htpv)
    batch = (w * HT_PER_W) // (htpf.shape[0] // (x2.shape[0] // L))
    httsidx = htpv[...] + batch * L
    pltpu.async_copy(tsf.at[httsidx], httsv, sem).wait()
    pltpu.async_copy(httsv, outts.at[htgv], sem).wait()


def _run_sc(x2, tsf, oidxf, htgf, htpf, token, B, L, D, R, n):
    info = plsc.get_sparse_core_info()
    NW = info.num_cores * info.num_subcores
    rows_per_w = (B * L) // NW
    nch = rows_per_w // _ROWS_CHUNK
    tch = rows_per_w // _TS_CHUNK
    HT_PER_W = (B * R) // NW

    tok_rows = jnp.broadcast_to(token[None, :], (HT_PER_W, D))
    mesh = plsc.VectorSubcoreMesh(core_axis_name="c", subcore_axis_name="s")

    kern = pl.kernel(
        functools.partial(_sc_body, L=L, D=D, n=n, NW=NW, HT_PER_W=HT_PER_W),
        out_type=(
            jax.ShapeDtypeStruct((B * n, D), jnp.float32),
            jax.ShapeDtypeStruct((B * n,), jnp.float32),
        ),
        mesh=mesh,
        scratch_types=(
            pltpu.VMEM((nch, _ROWS_CHUNK), jnp.int32),
            pltpu.VMEM((_ROWS_CHUNK, D), jnp.float32),
            pltpu.VMEM((tch, _TS_CHUNK), jnp.int32),
            pltpu.VMEM((tch, _TS_CHUNK), jnp.float32),
            pltpu.VMEM((HT_PER_W, D), jnp.float32),
            pltpu.VMEM((HT_PER_W,), jnp.int32),
            pltpu.VMEM((HT_PER_W,), jnp.int32),
            pltpu.VMEM((HT_PER_W,), jnp.float32),
            pltpu.SemaphoreType.DMA,
        ),
    )
    return kern(
        x2,
        tsf,
        tsf.reshape(-1, _TS_CHUNK),
        oidxf.reshape(-1, _ROWS_CHUNK),
        oidxf.reshape(-1, _TS_CHUNK),
        htgf,
        htpf,
        tok_rows,
    )


def kernel(x, timestamps, ht_positions, token):
    B, L, D = x.shape
    R = ht_positions.shape[1]
    n = L + R
    htp = ht_positions.astype(jnp.int32)

    oidx, htg, cq, isht = _run_meta(htp, B, L, R, n)
    mask = _run_mask(htp.reshape(B, 1, R), cq, isht, B, R, n, PR=128)

    new_x2, new_ts2 = _run_sc(
        x.reshape(B * L, D),
        timestamps.reshape(B * L),
        oidx.reshape(B * L),
        htg.reshape(B * R),
        htp.reshape(B * R),
        token,
        B, L, D, R, n,
    )
    return new_x2.reshape(B, n, D), new_ts2.reshape(B, n), mask.astype(jnp.bool_)


# speedup vs baseline: 6.4923x; 1.0246x over previous
import functools

import jax
import jax.numpy as jnp
from jax import lax
from jax.experimental import pallas as pl
from jax.experimental.pallas import tpu as pltpu
from jax.experimental.pallas import tpu_sc as plsc


def _meta_body(htp_smem, htp_ref, oidx_ref, htg_ref, cq_ref, isht_ref,
               *, L, R, n):
    b = pl.program_id(0)
    j_row = lax.broadcasted_iota(jnp.int32, (1, L), 1)
    q_row = lax.broadcasted_iota(jnp.int32, (1, n), 1)

    def body(i, carry):
        cnt_lt, c, isht = carry
        p = htp_smem[b, i]
        slot = p + i + 1
        cnt_lt = cnt_lt + (j_row > p).astype(jnp.int32)
        c = c + (q_row >= slot).astype(jnp.int32)
        isht = isht + (q_row == slot).astype(jnp.int32)
        return cnt_lt, c, isht

    zL = jnp.zeros((1, L), jnp.int32)
    zn = jnp.zeros((1, n), jnp.int32)
    cnt_lt, c, isht = lax.fori_loop(0, R, body, (zL, zn, zn))

    oidx_ref[0] = j_row + cnt_lt + b * n
    r_iota = lax.broadcasted_iota(jnp.int32, (1, R), 1)
    htg_ref[0] = htp_ref[0] + r_iota + 1 + b * n
    cq_ref[0] = c
    isht_ref[0] = isht


def _run_meta(htp, B, L, R, n):
    grid = (B,)
    out_shapes = (
        jax.ShapeDtypeStruct((B, 1, L), jnp.int32),
        jax.ShapeDtypeStruct((B, 1, R), jnp.int32),
        jax.ShapeDtypeStruct((B, 1, n), jnp.int32),
        jax.ShapeDtypeStruct((B, 1, n), jnp.int32),
    )
    return pl.pallas_call(
        functools.partial(_meta_body, L=L, R=R, n=n),
        grid=grid,
        in_specs=[
            pl.BlockSpec(memory_space=pltpu.SMEM),
            pl.BlockSpec((1, 1, R), lambda b: (b, 0, 0)),
        ],
        out_specs=(
            pl.BlockSpec((1, 1, L), lambda b: (b, 0, 0)),
            pl.BlockSpec((1, 1, R), lambda b: (b, 0, 0)),
            pl.BlockSpec((1, 1, n), lambda b: (b, 0, 0)),
            pl.BlockSpec((1, 1, n), lambda b: (b, 0, 0)),
        ),
        out_shape=out_shapes,
    )(htp, htp.reshape(B, 1, R))


def _mask_body(htp_ref, cq_ref, isht_ref, out_ref, *, R, n, PR):
    j = pl.program_id(1)
    r_iota = lax.broadcasted_iota(jnp.int32, (1, R), 1)
    htslot = htp_ref[0] + r_iota + 1
    p_col = lax.broadcasted_iota(jnp.int32, (PR, 1), 0) + j * PR
    le = htslot <= p_col
    c_p = jnp.sum(le.astype(jnp.int32), axis=1, keepdims=True)
    isht_p = jnp.sum((htslot == p_col).astype(jnp.int32), axis=1,
                     keepdims=True)
    H_p = jnp.max(jnp.where(le, jnp.broadcast_to(htslot, (PR, R)), 0),
                  axis=1, keepdims=True)
    H_p = jnp.where(isht_p > 0, 0, H_p)

    q_row = lax.broadcasted_iota(jnp.int32, (1, n), 1)
    c_q = cq_ref[0]
    isht_q = isht_ref[0]
    ne = jnp.minimum(jnp.abs(c_p - c_q), 1)
    lt = jnp.clip(H_p - q_row, 0, 1)
    out32 = isht_q * ne + (1 - isht_q) * lt
    out_ref[0] = out32.astype(jnp.int8)


def _run_mask(htp3, cq, isht, B, R, n, PR):
    grid = (B, n // PR)
    return pl.pallas_call(
        functools.partial(_mask_body, R=R, n=n, PR=PR),
        grid=grid,
        in_specs=[
            pl.BlockSpec((1, 1, R), lambda b, j: (b, 0, 0)),
            pl.BlockSpec((1, 1, n), lambda b, j: (b, 0, 0)),
            pl.BlockSpec((1, 1, n), lambda b, j: (b, 0, 0)),
        ],
        out_specs=pl.BlockSpec((1, PR, n), lambda b, j: (b, j, 0)),
        out_shape=jax.ShapeDtypeStruct((B, n, n), jnp.int8),
    )(htp3, cq, isht)


_ROWS_CHUNK = 16
_TS_CHUNK = 128
_NBUF = 4
_LA = 2


def _sc_body(x2, tsf, ts128, oidx32, oidx128, htg, htpf, tok_rows,
             outx2, outts, idxbuf, rowbuf, tsidx, tsbuf, tokbuf,
             htgv, htpv, httsv,
             si0, si1, si2, si3, so0, so1, so2, so3, sa,
             *, L, D, n, NW, HT_PER_W):
    semin = (si0, si1, si2, si3)
    semout = (so0, so1, so2, so3)
    CH = _ROWS_CHUNK
    c_i = lax.axis_index("c")
    s_i = lax.axis_index("s")
    w = s_i * 2 + c_i
    rows_per_w = x2.shape[0] // NW
    nch = rows_per_w // CH
    tch = rows_per_w // _TS_CHUNK
    wbase = w * rows_per_w
    batch = (w * HT_PER_W) // (htpf.shape[0] // (x2.shape[0] // L))

    aux = [
        pltpu.async_copy(oidx128.at[pl.ds(w * tch, tch)], tsidx, sa),
        pltpu.async_copy(ts128.at[pl.ds(w * tch, tch)], tsbuf, sa),
        pltpu.async_copy(htg.at[pl.ds(w * HT_PER_W, HT_PER_W)], htgv, sa),
        pltpu.async_copy(tok_rows, tokbuf, sa),
        pltpu.async_copy(htpf.at[pl.ds(w * HT_PER_W, HT_PER_W)], htpv, sa),
    ]
    pltpu.sync_copy(oidx32.at[pl.ds(w * nch, nch)], idxbuf)

    def start_in(f):
        s = f % _NBUF
        return pltpu.async_copy(x2.at[pl.ds(wbase + f * CH, CH)],
                                rowbuf.at[s], semin[s])

    ins, outs = {}, {}
    for f in range(min(_LA, nch)):
        ins[f] = start_in(f)
    for j in range(nch):
        s = j % _NBUF
        f = j + _LA
        if f < nch:
            fs = f % _NBUF
            if f >= _NBUF:
                outs[f - _NBUF].wait()
            ins[f] = start_in(f)
        ins[j].wait()
        outs[j] = pltpu.async_copy(rowbuf.at[s], outx2.at[idxbuf.at[j]],
                                   semout[s])
    for j in range(max(0, nch - _NBUF), nch):
        outs[j].wait()

    for cp in aux:
        cp.wait()
    tsc = [pltpu.async_copy(tsbuf.at[j], outts.at[tsidx.at[j]], sa)
           for j in range(tch)]

    tokc = pltpu.async_copy(tokbuf, outx2.at[htgv], sa)
    httsidx = htpv[...] + batch * L
    pltpu.async_copy(tsf.at[httsidx], httsv, si0).wait()
    pltpu.async_copy(httsv, outts.at[htgv], sa).wait()
    for cp in tsc:
        cp.wait()
    tokc.wait()


def _run_sc(x2, tsf, oidxf, htgf, htpf, token, B, L, D, R, n):
    info = plsc.get_sparse_core_info()
    NW = info.num_cores * info.num_subcores
    rows_per_w = (B * L) // NW
    nch = rows_per_w // _ROWS_CHUNK
    tch = rows_per_w // _TS_CHUNK
    HT_PER_W = (B * R) // NW

    tok_rows = jnp.broadcast_to(token[None, :], (HT_PER_W, D))
    mesh = plsc.VectorSubcoreMesh(core_axis_name="c", subcore_axis_name="s")

    kern = pl.kernel(
        functools.partial(_sc_body, L=L, D=D, n=n, NW=NW, HT_PER_W=HT_PER_W),
        out_type=(
            jax.ShapeDtypeStruct((B * n, D), jnp.float32),
            jax.ShapeDtypeStruct((B * n,), jnp.float32),
        ),
        mesh=mesh,
        scratch_types=(
            pltpu.VMEM((nch, _ROWS_CHUNK), jnp.int32),
            pltpu.VMEM((_NBUF, _ROWS_CHUNK, D), jnp.float32),
            pltpu.VMEM((tch, _TS_CHUNK), jnp.int32),
            pltpu.VMEM((tch, _TS_CHUNK), jnp.float32),
            pltpu.VMEM((HT_PER_W, D), jnp.float32),
            pltpu.VMEM((HT_PER_W,), jnp.int32),
            pltpu.VMEM((HT_PER_W,), jnp.int32),
            pltpu.VMEM((HT_PER_W,), jnp.float32),
        ) + (pltpu.SemaphoreType.DMA,) * 9,
    )
    return kern(
        x2,
        tsf,
        tsf.reshape(-1, _TS_CHUNK),
        oidxf.reshape(-1, _ROWS_CHUNK),
        oidxf.reshape(-1, _TS_CHUNK),
        htgf,
        htpf,
        tok_rows,
    )


def kernel(x, timestamps, ht_positions, token):
    B, L, D = x.shape
    R = ht_positions.shape[1]
    n = L + R
    htp = ht_positions.astype(jnp.int32)

    oidx, htg, cq, isht = _run_meta(htp, B, L, R, n)
    mask = _run_mask(htp.reshape(B, 1, R), cq, isht, B, R, n, PR=128)

    new_x2, new_ts2 = _run_sc(
        x.reshape(B * L, D),
        timestamps.reshape(B * L),
        oidx.reshape(B * L),
        htg.reshape(B * R),
        htp.reshape(B * R),
        token,
        B, L, D, R, n,
    )
    return new_x2.reshape(B, n, D), new_ts2.reshape(B, n), mask.astype(jnp.bool_)


# speedup vs baseline: 6.4934x; 1.0002x over previous
import functools

import jax
import jax.numpy as jnp
from jax import lax
from jax.experimental import pallas as pl
from jax.experimental.pallas import tpu as pltpu
from jax.experimental.pallas import tpu_sc as plsc


def _meta_body(htp_smem, htp_ref, oidx_ref, htg_ref, cq_ref, isht_ref,
               *, L, R, n):
    b = pl.program_id(0)
    j_row = lax.broadcasted_iota(jnp.int32, (1, L), 1)
    q_row = lax.broadcasted_iota(jnp.int32, (1, n), 1)

    def body(i, carry):
        cnt_lt, c, isht = carry
        p = htp_smem[b, i]
        slot = p + i + 1
        cnt_lt = cnt_lt + (j_row > p).astype(jnp.int32)
        c = c + (q_row >= slot).astype(jnp.int32)
        isht = isht + (q_row == slot).astype(jnp.int32)
        return cnt_lt, c, isht

    zL = jnp.zeros((1, L), jnp.int32)
    zn = jnp.zeros((1, n), jnp.int32)
    cnt_lt, c, isht = lax.fori_loop(0, R, body, (zL, zn, zn))

    oidx_ref[0] = j_row + cnt_lt + b * n
    r_iota = lax.broadcasted_iota(jnp.int32, (1, R), 1)
    htg_ref[0] = htp_ref[0] + r_iota + 1 + b * n
    cq_ref[0] = c
    isht_ref[0] = isht


def _run_meta(htp, B, L, R, n):
    grid = (B,)
    out_shapes = (
        jax.ShapeDtypeStruct((B, 1, L), jnp.int32),
        jax.ShapeDtypeStruct((B, 1, R), jnp.int32),
        jax.ShapeDtypeStruct((B, 1, n), jnp.int32),
        jax.ShapeDtypeStruct((B, 1, n), jnp.int32),
    )
    return pl.pallas_call(
        functools.partial(_meta_body, L=L, R=R, n=n),
        grid=grid,
        in_specs=[
            pl.BlockSpec(memory_space=pltpu.SMEM),
            pl.BlockSpec((1, 1, R), lambda b: (b, 0, 0)),
        ],
        out_specs=(
            pl.BlockSpec((1, 1, L), lambda b: (b, 0, 0)),
            pl.BlockSpec((1, 1, R), lambda b: (b, 0, 0)),
            pl.BlockSpec((1, 1, n), lambda b: (b, 0, 0)),
            pl.BlockSpec((1, 1, n), lambda b: (b, 0, 0)),
        ),
        out_shape=out_shapes,
    )(htp, htp.reshape(B, 1, R))


def _mask_body(htp_ref, cq_ref, isht_ref, out_ref, *, R, n, PR):
    j = pl.program_id(1)
    r_iota = lax.broadcasted_iota(jnp.int32, (1, R), 1)
    htslot = htp_ref[0] + r_iota + 1
    p_col = lax.broadcasted_iota(jnp.int32, (PR, 1), 0) + j * PR
    le = htslot <= p_col
    c_p = jnp.sum(le.astype(jnp.int32), axis=1, keepdims=True)
    isht_p = jnp.sum((htslot == p_col).astype(jnp.int32), axis=1,
                     keepdims=True)
    H_p = jnp.max(jnp.where(le, jnp.broadcast_to(htslot, (PR, R)), 0),
                  axis=1, keepdims=True)
    H_p = jnp.where(isht_p > 0, 0, H_p)

    q_row = lax.broadcasted_iota(jnp.int32, (1, n), 1)
    c_q = cq_ref[0]
    isht_q = isht_ref[0]
    ne = jnp.minimum(jnp.abs(c_p - c_q), 1)
    lt = jnp.clip(H_p - q_row, 0, 1)
    out32 = isht_q * ne + (1 - isht_q) * lt
    out_ref[0] = out32.astype(jnp.int8)


def _run_mask(htp3, cq, isht, B, R, n, PR):
    grid = (B, n // PR)
    return pl.pallas_call(
        functools.partial(_mask_body, R=R, n=n, PR=PR),
        grid=grid,
        in_specs=[
            pl.BlockSpec((1, 1, R), lambda b, j: (b, 0, 0)),
            pl.BlockSpec((1, 1, n), lambda b, j: (b, 0, 0)),
            pl.BlockSpec((1, 1, n), lambda b, j: (b, 0, 0)),
        ],
        out_specs=pl.BlockSpec((1, PR, n), lambda b, j: (b, j, 0)),
        out_shape=jax.ShapeDtypeStruct((B, n, n), jnp.int8),
    )(htp3, cq, isht)


_ROWS_CHUNK = 16
_TS_CHUNK = 128
_NBUF = 4
_LA = 2


def _sc_body(x2, tsf, ts128, oidx32, oidx128, htg, htpf, tok_rows,
             outx2, outts, idxbuf, rowbuf, tsidx, tsbuf, tokbuf,
             htgv, htpv, httsv,
             si0, si1, si2, si3, so0, so1, so2, so3, sa,
             *, L, D, n, NW, HT_PER_W):
    semin = (si0, si1, si2, si3)
    semout = (so0, so1, so2, so3)
    CH = _ROWS_CHUNK
    c_i = lax.axis_index("c")
    s_i = lax.axis_index("s")
    w = s_i * 2 + c_i
    rows_per_w = x2.shape[0] // NW
    nch = rows_per_w // CH
    tch = rows_per_w // _TS_CHUNK
    wbase = w * rows_per_w
    batch = (w * HT_PER_W) // (htpf.shape[0] // (x2.shape[0] // L))

    aux = [
        pltpu.async_copy(oidx128.at[pl.ds(w * tch, tch)], tsidx, sa),
        pltpu.async_copy(ts128.at[pl.ds(w * tch, tch)], tsbuf, sa),
        pltpu.async_copy(htg.at[pl.ds(w * HT_PER_W, HT_PER_W)], htgv, sa),
        pltpu.async_copy(tok_rows, tokbuf, sa),
        pltpu.async_copy(htpf.at[pl.ds(w * HT_PER_W, HT_PER_W)], htpv, sa),
    ]
    pltpu.sync_copy(oidx32.at[pl.ds(w * nch, nch)], idxbuf)

    def start_in(f):
        s = f % _NBUF
        return pltpu.async_copy(x2.at[pl.ds(wbase + f * CH, CH)],
                                rowbuf.at[s], semin[s])

    ins, outs = {}, {}
    for f in range(min(_LA, nch)):
        ins[f] = start_in(f)
    for j in range(nch):
        s = j % _NBUF
        f = j + _LA
        if f < nch:
            fs = f % _NBUF
            if f >= _NBUF:
                outs[f - _NBUF].wait()
            ins[f] = start_in(f)
        ins[j].wait()
        outs[j] = pltpu.async_copy(rowbuf.at[s], outx2.at[idxbuf.at[j]],
                                   semout[s])
    for j in range(max(0, nch - _NBUF), nch):
        outs[j].wait()

    for cp in aux:
        cp.wait()
    tsc = [pltpu.async_copy(tsbuf.at[j], outts.at[tsidx.at[j]], sa)
           for j in range(tch)]

    tokc = pltpu.async_copy(tokbuf, outx2.at[htgv], sa)
    httsidx = htpv[...] + batch * L
    pltpu.async_copy(tsf.at[httsidx], httsv, si0).wait()
    pltpu.async_copy(httsv, outts.at[htgv], sa).wait()
    for cp in tsc:
        cp.wait()
    tokc.wait()


def _run_sc(x2, tsf, oidxf, htgf, htpf, token, B, L, D, R, n):
    info = plsc.get_sparse_core_info()
    NW = info.num_cores * info.num_subcores
    rows_per_w = (B * L) // NW
    nch = rows_per_w // _ROWS_CHUNK
    tch = rows_per_w // _TS_CHUNK
    HT_PER_W = (B * R) // NW

    tok_rows = jnp.broadcast_to(token[None, :], (HT_PER_W, D))
    mesh = plsc.VectorSubcoreMesh(core_axis_name="c", subcore_axis_name="s")

    kern = pl.kernel(
        functools.partial(_sc_body, L=L, D=D, n=n, NW=NW, HT_PER_W=HT_PER_W),
        out_type=(
            jax.ShapeDtypeStruct((B * n, D), jnp.float32),
            jax.ShapeDtypeStruct((B * n,), jnp.float32),
        ),
        mesh=mesh,
        scratch_types=(
            pltpu.VMEM((nch, _ROWS_CHUNK), jnp.int32),
            pltpu.VMEM((_NBUF, _ROWS_CHUNK, D), jnp.float32),
            pltpu.VMEM((tch, _TS_CHUNK), jnp.int32),
            pltpu.VMEM((tch, _TS_CHUNK), jnp.float32),
            pltpu.VMEM((HT_PER_W, D), jnp.float32),
            pltpu.VMEM((HT_PER_W,), jnp.int32),
            pltpu.VMEM((HT_PER_W,), jnp.int32),
            pltpu.VMEM((HT_PER_W,), jnp.float32),
        ) + (pltpu.SemaphoreType.DMA,) * 9,
    )
    return kern(
        x2,
        tsf,
        tsf.reshape(-1, _TS_CHUNK),
        oidxf.reshape(-1, _ROWS_CHUNK),
        oidxf.reshape(-1, _TS_CHUNK),
        htgf,
        htpf,
        tok_rows,
    )


def kernel(x, timestamps, ht_positions, token):
    B, L, D = x.shape
    R = ht_positions.shape[1]
    n = L + R
    htp = ht_positions.astype(jnp.int32)

    oidx, htg, cq, isht = _run_meta(htp, B, L, R, n)
    new_x2, new_ts2 = _run_sc(
        x.reshape(B * L, D),
        timestamps.reshape(B * L),
        oidx.reshape(B * L),
        htg.reshape(B * R),
        htp.reshape(B * R),
        token,
        B, L, D, R, n,
    )
    mask = _run_mask(htp.reshape(B, 1, R), cq, isht, B, R, n, PR=128)
    return new_x2.reshape(B, n, D), new_ts2.reshape(B, n), mask.astype(jnp.bool_)


# speedup vs baseline: 7.2798x; 1.1211x over previous
import functools

import jax
import jax.numpy as jnp
from jax import lax
from jax.experimental import pallas as pl
from jax.experimental.pallas import tpu as pltpu
from jax.experimental.pallas import tpu_sc as plsc


def _meta_body(htpc_ref, htpr_ref, oidx_ref, htg_ref, cq_ref, isht_ref,
               *, L, R, n):
    b = pl.program_id(0)
    htp_col = htpc_ref[0]
    iota_col = lax.broadcasted_iota(jnp.int32, (R, 1), 0)
    slot_col = htp_col + iota_col + 1
    j_row = lax.broadcasted_iota(jnp.int32, (1, L), 1)
    q_row = lax.broadcasted_iota(jnp.int32, (1, n), 1)

    cnt_lt = jnp.sum((htp_col < j_row).astype(jnp.int32), axis=0,
                     keepdims=True)
    c = jnp.sum((slot_col <= q_row).astype(jnp.int32), axis=0,
                keepdims=True)
    isht = jnp.sum((slot_col == q_row).astype(jnp.int32), axis=0,
                   keepdims=True)

    oidx_ref[0] = j_row + cnt_lt + b * n
    r_iota = lax.broadcasted_iota(jnp.int32, (1, R), 1)
    htg_ref[0] = htpr_ref[0] + r_iota + 1 + b * n
    cq_ref[0] = c
    isht_ref[0] = isht


def _run_meta(htp, B, L, R, n):
    grid = (B,)
    out_shapes = (
        jax.ShapeDtypeStruct((B, 1, L), jnp.int32),
        jax.ShapeDtypeStruct((B, 1, R), jnp.int32),
        jax.ShapeDtypeStruct((B, 1, n), jnp.int32),
        jax.ShapeDtypeStruct((B, 1, n), jnp.int32),
    )
    return pl.pallas_call(
        functools.partial(_meta_body, L=L, R=R, n=n),
        grid=grid,
        in_specs=[
            pl.BlockSpec((1, R, 1), lambda b: (b, 0, 0)),
            pl.BlockSpec((1, 1, R), lambda b: (b, 0, 0)),
        ],
        out_specs=(
            pl.BlockSpec((1, 1, L), lambda b: (b, 0, 0)),
            pl.BlockSpec((1, 1, R), lambda b: (b, 0, 0)),
            pl.BlockSpec((1, 1, n), lambda b: (b, 0, 0)),
            pl.BlockSpec((1, 1, n), lambda b: (b, 0, 0)),
        ),
        out_shape=out_shapes,
    )(htp.reshape(B, R, 1), htp.reshape(B, 1, R))


def _mask_body(htp_ref, cq_ref, isht_ref, out_ref, *, R, n, PR):
    j = pl.program_id(1)
    r_iota = lax.broadcasted_iota(jnp.int32, (1, R), 1)
    htslot = htp_ref[0] + r_iota + 1
    p_col = lax.broadcasted_iota(jnp.int32, (PR, 1), 0) + j * PR
    le = htslot <= p_col
    T_p = jnp.max(jnp.where(le, jnp.broadcast_to(htslot, (PR, R)), 0),
                  axis=1, keepdims=True)
    isht_p = jnp.max((htslot == p_col).astype(jnp.int32), axis=1,
                     keepdims=True)
    H2_p = jnp.where(isht_p > 0, 0, T_p)

    q_row = lax.broadcasted_iota(jnp.int32, (1, n), 1)
    isht_b = isht_ref[0] != 0
    m = (q_row < H2_p) | ((q_row != T_p) & isht_b)
    ones = jnp.minimum(cq_ref[0] + 1, 1)
    out_ref[0] = (m.astype(jnp.int32) * ones).astype(jnp.int8)


def _run_mask(htp3, cq, isht, B, R, n, PR):
    grid = (B, n // PR)
    return pl.pallas_call(
        functools.partial(_mask_body, R=R, n=n, PR=PR),
        grid=grid,
        in_specs=[
            pl.BlockSpec((1, 1, R), lambda b, j: (b, 0, 0)),
            pl.BlockSpec((1, 1, n), lambda b, j: (b, 0, 0)),
            pl.BlockSpec((1, 1, n), lambda b, j: (b, 0, 0)),
        ],
        out_specs=pl.BlockSpec((1, PR, n), lambda b, j: (b, j, 0)),
        out_shape=jax.ShapeDtypeStruct((B, n, n), jnp.int8),
    )(htp3, cq, isht)


_ROWS_CHUNK = 32
_TS_CHUNK = 128
_NBUF = 3
_LA = 1


def _sc_body(x2, tsf, ts128, oidx32, oidx128, htg, htpf, tok_rows,
             outx2, outts, idxbuf, rowbuf, tsidx, tsbuf, tokbuf,
             htgv, htpv, httsv,
             si0, si1, si2, si3, so0, so1, so2, so3, sa,
             *, L, D, n, NW, HT_PER_W):
    semin = (si0, si1, si2, si3)[:_NBUF]
    semout = (so0, so1, so2, so3)[:_NBUF]
    CH = _ROWS_CHUNK
    c_i = lax.axis_index("c")
    s_i = lax.axis_index("s")
    w = s_i * 2 + c_i
    rows_per_w = x2.shape[0] // NW
    nch = rows_per_w // CH
    tch = rows_per_w // _TS_CHUNK
    wbase = w * rows_per_w
    batch = (w * HT_PER_W) // (htpf.shape[0] // (x2.shape[0] // L))

    aux = [
        pltpu.async_copy(oidx128.at[pl.ds(w * tch, tch)], tsidx, sa),
        pltpu.async_copy(ts128.at[pl.ds(w * tch, tch)], tsbuf, sa),
        pltpu.async_copy(htg.at[pl.ds(w * HT_PER_W, HT_PER_W)], htgv, sa),
        pltpu.async_copy(tok_rows, tokbuf, sa),
        pltpu.async_copy(htpf.at[pl.ds(w * HT_PER_W, HT_PER_W)], htpv, sa),
    ]
    pltpu.sync_copy(oidx32.at[pl.ds(w * nch, nch)], idxbuf)

    def start_in(f):
        s = f % _NBUF
        return pltpu.async_copy(x2.at[pl.ds(wbase + f * CH, CH)],
                                rowbuf.at[s], semin[s])

    ins, outs = {}, {}
    for f in range(min(_LA, nch)):
        ins[f] = start_in(f)
    for j in range(nch):
        s = j % _NBUF
        f = j + _LA
        if f < nch:
            fs = f % _NBUF
            if f >= _NBUF:
                outs[f - _NBUF].wait()
            ins[f] = start_in(f)
        ins[j].wait()
        outs[j] = pltpu.async_copy(rowbuf.at[s], outx2.at[idxbuf.at[j]],
                                   semout[s])
    for j in range(max(0, nch - _NBUF), nch):
        outs[j].wait()

    for cp in aux:
        cp.wait()
    tsc = [pltpu.async_copy(tsbuf.at[j], outts.at[tsidx.at[j]], sa)
           for j in range(tch)]

    tokc = pltpu.async_copy(tokbuf, outx2.at[htgv], sa)
    httsidx = htpv[...] + batch * L
    pltpu.async_copy(tsf.at[httsidx], httsv, si0).wait()
    pltpu.async_copy(httsv, outts.at[htgv], sa).wait()
    for cp in tsc:
        cp.wait()
    tokc.wait()


def _run_sc(x2, tsf, oidxf, htgf, htpf, token, B, L, D, R, n):
    info = plsc.get_sparse_core_info()
    NW = info.num_cores * info.num_subcores
    rows_per_w = (B * L) // NW
    nch = rows_per_w // _ROWS_CHUNK
    tch = rows_per_w // _TS_CHUNK
    HT_PER_W = (B * R) // NW

    tok_rows = jnp.broadcast_to(token[None, :], (HT_PER_W, D))
    mesh = plsc.VectorSubcoreMesh(core_axis_name="c", subcore_axis_name="s")

    kern = pl.kernel(
        functools.partial(_sc_body, L=L, D=D, n=n, NW=NW, HT_PER_W=HT_PER_W),
        out_type=(
            jax.ShapeDtypeStruct((B * n, D), jnp.float32),
            jax.ShapeDtypeStruct((B * n,), jnp.float32),
        ),
        mesh=mesh,
        scratch_types=(
            pltpu.VMEM((nch, _ROWS_CHUNK), jnp.int32),
            pltpu.VMEM((_NBUF, _ROWS_CHUNK, D), jnp.float32),
            pltpu.VMEM((tch, _TS_CHUNK), jnp.int32),
            pltpu.VMEM((tch, _TS_CHUNK), jnp.float32),
            pltpu.VMEM((HT_PER_W, D), jnp.float32),
            pltpu.VMEM((HT_PER_W,), jnp.int32),
            pltpu.VMEM((HT_PER_W,), jnp.int32),
            pltpu.VMEM((HT_PER_W,), jnp.float32),
        ) + (pltpu.SemaphoreType.DMA,) * 9,
    )
    return kern(
        x2,
        tsf,
        tsf.reshape(-1, _TS_CHUNK),
        oidxf.reshape(-1, _ROWS_CHUNK),
        oidxf.reshape(-1, _TS_CHUNK),
        htgf,
        htpf,
        tok_rows,
    )


def kernel(x, timestamps, ht_positions, token):
    B, L, D = x.shape
    R = ht_positions.shape[1]
    n = L + R
    htp = ht_positions.astype(jnp.int32)

    oidx, htg, cq, isht = _run_meta(htp, B, L, R, n)
    new_x2, new_ts2 = _run_sc(
        x.reshape(B * L, D),
        timestamps.reshape(B * L),
        oidx.reshape(B * L),
        htg.reshape(B * R),
        htp.reshape(B * R),
        token,
        B, L, D, R, n,
    )
    mask = _run_mask(htp.reshape(B, 1, R), cq, isht, B, R, n, PR=128)
    return new_x2.reshape(B, n, D), new_ts2.reshape(B, n), mask.view(jnp.bool_)
